# Initial kernel scaffold; baseline (speedup 1.0000x reference)
import jax
import jax.numpy as jnp
from jax.experimental import pallas as pl


def kernel(center, positive, negative, target_embeddings, context_embeddings):
    raise NotImplementedError("write your pallas kernel here")


# speedup vs baseline: 3.0984x; 3.0984x over previous
import functools

import jax
import jax.numpy as jnp
from jax import lax
from jax.experimental import pallas as pl
from jax.experimental.pallas import tpu as pltpu
from jax.experimental.pallas import tpu_sc as plsc

_B = 4096
_P = 50
_N = 200
_D = 128
_R = 256
_L = 16
_NC = 2
_NS = 16
_NW = _NC * _NS
_BW = _B // _NW
_G = _R // _L


def _sc_body(cidx_hbm, ctx_idx_hbm, tgt_hbm, ctx_hbm, out_hbm,
             cidx_v, ctr_v, idx_v, rows_v, sc_v, tmp_v,
             sem_c, sem_r0, sem_r1, sem_s0, sem_s1):
    wid = lax.axis_index("s") * _NC + lax.axis_index("c")
    base = pl.multiple_of(wid * _BW, 8)

    pltpu.sync_copy(cidx_hbm.at[pl.ds(base, _BW)], cidx_v)
    pltpu.sync_copy(ctx_idx_hbm.at[pl.ds(pl.multiple_of(wid * _BW * 2, 8), _BW * 2)],
                    idx_v)
    pltpu.async_copy(tgt_hbm.at[cidx_v], ctr_v, sem_c).wait()

    row_sems = (sem_r0, sem_r1)
    sc_sems = (sem_s0, sem_s1)
    iota = lax.iota(jnp.int32, _L)
    iota16 = iota * _L

    def issue_gather(b, buf):
        for h in range(2):
            pltpu.async_copy(ctx_hbm.at[idx_v.at[b * 2 + h]],
                             rows_v.at[buf, pl.ds(h * 128, 128)],
                             row_sems[buf])

    def wait_gather(buf):
        pltpu.make_async_copy(ctx_hbm.at[pl.ds(0, _R)], rows_v.at[buf],
                              row_sems[buf]).wait()

    issue_gather(0, 0)
    issue_gather(1, 1)

    def step(i, carry):
        for u in range(2):
            b = i * 2 + u
            buf = u
            wait_gather(buf)

            @pl.when(b >= 2)
            def _wait_prev_scores():
                pltpu.make_async_copy(sc_v.at[buf], out_hbm.at[pl.ds(0, _R)],
                                      sc_sems[buf]).wait()

            ctr = [ctr_v[b, pl.ds(c * _L, _L)] for c in range(_D // _L)]
            rv = rows_v.at[buf]

            def group(g, carry2):
                for r in range(_L):
                    row = g * _L + r
                    acc = rv[row, pl.ds(0, _L)] * ctr[0]
                    for c in range(1, _D // _L):
                        acc = acc + rv[row, pl.ds(c * _L, _L)] * ctr[c]
                    tmp_v[pl.ds(r * _L, _L)] = acc
                svec = plsc.load_gather(tmp_v, [iota16])
                for k in range(1, _L):
                    svec = svec + plsc.load_gather(tmp_v, [iota16 + k])
                sc_v[buf, pl.ds(pl.multiple_of(g * _L, 8), _L)] = svec
                return carry2

            lax.fori_loop(0, _G, group, 0)

            pltpu.async_copy(
                sc_v.at[buf],
                out_hbm.at[pl.ds(pl.multiple_of((base + b) * _R, 8), _R)],
                sc_sems[buf])

            @pl.when(b + 2 < _BW)
            def _refill():
                issue_gather(b + 2, buf)
        return carry

    lax.fori_loop(0, _BW // 2, step, 0)

    for buf in range(2):
        pltpu.make_async_copy(sc_v.at[buf], out_hbm.at[pl.ds(0, _R)],
                              sc_sems[buf]).wait()


@jax.jit
def _sc_scores(cidx, ctx_idx, tgt, ctx):
    mesh = plsc.VectorSubcoreMesh(core_axis_name="c", subcore_axis_name="s",
                                  num_cores=_NC, num_subcores=_NS)
    f = pl.kernel(
        _sc_body,
        out_type=jax.ShapeDtypeStruct((_B * _R,), jnp.float32),
        mesh=mesh,
        compiler_params=pltpu.CompilerParams(needs_layout_passes=False),
        scratch_types=[
            pltpu.VMEM((_BW,), jnp.int32),
            pltpu.VMEM((_BW, _D), jnp.float32),
            pltpu.VMEM((_BW * 2, 128), jnp.int32),
            pltpu.VMEM((2, _R, _D), jnp.float32),
            pltpu.VMEM((2, _R), jnp.float32),
            pltpu.VMEM((_L * _L,), jnp.float32),
            pltpu.SemaphoreType.DMA,
            pltpu.SemaphoreType.DMA,
            pltpu.SemaphoreType.DMA,
            pltpu.SemaphoreType.DMA,
            pltpu.SemaphoreType.DMA,
        ],
    )
    return f(cidx, ctx_idx, tgt, ctx)


def kernel(center, positive, negative, target_embeddings, context_embeddings):
    center = center.astype(jnp.int32)
    pad = jnp.zeros((_B, _R - _P - _N), jnp.int32)
    ctx_idx = jnp.concatenate(
        [positive.astype(jnp.int32), negative.astype(jnp.int32), pad], axis=1)
    ctx_idx = ctx_idx.reshape(_B * 2, 128)
    scores = _sc_scores(center, ctx_idx, target_embeddings, context_embeddings)
    scores = scores.reshape(_B, _R)
    return scores[:, :_P], scores[:, _P:_P + _N]


# speedup vs baseline: 3.1128x; 1.0046x over previous
import functools

import jax
import jax.numpy as jnp
from jax import lax
from jax.experimental import pallas as pl
from jax.experimental.pallas import tpu as pltpu
from jax.experimental.pallas import tpu_sc as plsc

_B = 4096
_P = 50
_N = 200
_D = 128
_R = 256
_L = 16
_NC = 2
_NS = 16
_NW = _NC * _NS
_BW = _B // _NW
_G = _R // _L


def _sc_body(cidx_hbm, ctx_idx_hbm, tgt_hbm, ctx_hbm, out_hbm,
             cidx_v, ctr_v, idx_v, rows_v, sc_v,
             sem_c, sem_r0, sem_r1, sem_s0, sem_s1):
    wid = lax.axis_index("s") * _NC + lax.axis_index("c")
    base = pl.multiple_of(wid * _BW, 8)

    pltpu.sync_copy(cidx_hbm.at[pl.ds(base, _BW)], cidx_v)
    pltpu.sync_copy(ctx_idx_hbm.at[pl.ds(pl.multiple_of(wid * _BW * 2, 8), _BW * 2)],
                    idx_v)
    pltpu.async_copy(tgt_hbm.at[cidx_v], ctr_v, sem_c).wait()

    row_sems = (sem_r0, sem_r1)
    sc_sems = (sem_s0, sem_s1)
    iota = lax.iota(jnp.int32, _L)
    perms = [jnp.bitwise_xor(iota, k) for k in (8, 4, 2, 1)]

    dnums = lax.GatherDimensionNumbers(
        offset_dims=(), collapsed_slice_dims=(0,), start_index_map=(0,))

    def vperm(x, idx):
        return lax.gather(x, idx[:, None], dnums, (1,),
                          mode=lax.GatherScatterMode.PROMISE_IN_BOUNDS)

    def issue_gather(b, buf):
        for h in range(2):
            pltpu.async_copy(ctx_hbm.at[idx_v.at[b * 2 + h]],
                             rows_v.at[buf, pl.ds(h * 128, 128)],
                             row_sems[buf])

    def wait_gather(buf):
        pltpu.make_async_copy(ctx_hbm.at[pl.ds(0, _R)], rows_v.at[buf],
                              row_sems[buf]).wait()

    issue_gather(0, 0)
    issue_gather(1, 1)

    def step(i, carry):
        for u in range(2):
            b = i * 2 + u
            buf = u
            wait_gather(buf)

            @pl.when(b >= 2)
            def _wait_prev_scores():
                pltpu.make_async_copy(sc_v.at[buf], out_hbm.at[pl.ds(0, _R)],
                                      sc_sems[buf]).wait()

            ctr = [ctr_v[b, pl.ds(c * _L, _L)] for c in range(_D // _L)]
            rv = rows_v.at[buf]

            def group(g, carry2):
                svec = None
                for r in range(_L):
                    row = g * _L + r
                    prods = [rv[row, pl.ds(c * _L, _L)] * ctr[c]
                             for c in range(_D // _L)]
                    while len(prods) > 1:
                        prods = [prods[i] + prods[i + 1]
                                 for i in range(0, len(prods), 2)]
                    s = prods[0]
                    for p in perms:
                        s = s + vperm(s, p)
                    svec = s if svec is None else jnp.where(iota == r, s, svec)
                sc_v[buf, pl.ds(pl.multiple_of(g * _L, 8), _L)] = svec
                return carry2

            lax.fori_loop(0, _G, group, 0)

            pltpu.async_copy(
                sc_v.at[buf],
                out_hbm.at[pl.ds(pl.multiple_of((base + b) * _R, 8), _R)],
                sc_sems[buf])

            @pl.when(b + 2 < _BW)
            def _refill():
                issue_gather(b + 2, buf)
        return carry

    lax.fori_loop(0, _BW // 2, step, 0)

    for buf in range(2):
        pltpu.make_async_copy(sc_v.at[buf], out_hbm.at[pl.ds(0, _R)],
                              sc_sems[buf]).wait()


@jax.jit
def _sc_scores(cidx, ctx_idx, tgt, ctx):
    mesh = plsc.VectorSubcoreMesh(core_axis_name="c", subcore_axis_name="s",
                                  num_cores=_NC, num_subcores=_NS)
    f = pl.kernel(
        _sc_body,
        out_type=jax.ShapeDtypeStruct((_B * _R,), jnp.float32),
        mesh=mesh,
        compiler_params=pltpu.CompilerParams(needs_layout_passes=False),
        scratch_types=[
            pltpu.VMEM((_BW,), jnp.int32),
            pltpu.VMEM((_BW, _D), jnp.float32),
            pltpu.VMEM((_BW * 2, 128), jnp.int32),
            pltpu.VMEM((2, _R, _D), jnp.float32),
            pltpu.VMEM((2, _R), jnp.float32),
            pltpu.SemaphoreType.DMA,
            pltpu.SemaphoreType.DMA,
            pltpu.SemaphoreType.DMA,
            pltpu.SemaphoreType.DMA,
            pltpu.SemaphoreType.DMA,
        ],
    )
    return f(cidx, ctx_idx, tgt, ctx)


def kernel(center, positive, negative, target_embeddings, context_embeddings):
    center = center.astype(jnp.int32)
    pad = jnp.zeros((_B, _R - _P - _N), jnp.int32)
    ctx_idx = jnp.concatenate(
        [positive.astype(jnp.int32), negative.astype(jnp.int32), pad], axis=1)
    ctx_idx = ctx_idx.reshape(_B * 2, 128)
    scores = _sc_scores(center, ctx_idx, target_embeddings, context_embeddings)
    scores = scores.reshape(_B, _R)
    return scores[:, :_P], scores[:, _P:_P + _N]


# speedup vs baseline: 3.3624x; 1.0802x over previous
import functools

import jax
import jax.numpy as jnp
from jax import lax
from jax.experimental import pallas as pl
from jax.experimental.pallas import tpu as pltpu
from jax.experimental.pallas import tpu_sc as plsc

_B = 4096
_P = 50
_N = 200
_D = 128
_R = 256
_L = 16
_NC = 2
_NS = 16
_NW = _NC * _NS
_BW = _B // _NW
_G = _R // _L


def _sc_body(cidx_hbm, ctx_idx_hbm, tgt_hbm, ctx_hbm, out_hbm,
             cidx_v, ctr_v, idx_v, rows_v, sc_v,
             sem_c, sem_r0, sem_r1, sem_s0, sem_s1):
    wid = lax.axis_index("s") * _NC + lax.axis_index("c")
    base = pl.multiple_of(wid * _BW, 8)

    pltpu.sync_copy(cidx_hbm.at[pl.ds(base, _BW)], cidx_v)
    pltpu.sync_copy(ctx_idx_hbm.at[pl.ds(pl.multiple_of(wid * _BW * 2, 8), _BW * 2)],
                    idx_v)
    pltpu.async_copy(tgt_hbm.at[cidx_v], ctr_v, sem_c).wait()

    row_sems = (sem_r0, sem_r1)
    sc_sems = (sem_s0, sem_s1)
    iota = lax.iota(jnp.int32, _L)
    perms = [jnp.bitwise_xor(iota, k) for k in (8, 4, 2, 1)]

    dnums = lax.GatherDimensionNumbers(
        offset_dims=(), collapsed_slice_dims=(0,), start_index_map=(0,))

    def vperm(x, idx):
        return lax.gather(x, idx[:, None], dnums, (1,),
                          mode=lax.GatherScatterMode.PROMISE_IN_BOUNDS)

    def issue_gather(b, buf):
        for h in range(2):
            pltpu.async_copy(ctx_hbm.at[idx_v.at[b * 2 + h]],
                             rows_v.at[buf, pl.ds(h * 128, 128)],
                             row_sems[buf])

    def wait_gather(buf):
        pltpu.make_async_copy(ctx_hbm.at[pl.ds(0, _R)], rows_v.at[buf],
                              row_sems[buf]).wait()

    issue_gather(0, 0)
    issue_gather(1, 1)

    def step(i, carry):
        for u in range(2):
            b = i * 2 + u
            buf = u
            wait_gather(buf)

            @pl.when(b >= 2)
            def _wait_prev_scores():
                pltpu.make_async_copy(sc_v.at[buf], out_hbm.at[pl.ds(0, _R)],
                                      sc_sems[buf]).wait()

            ctr = [ctr_v[b, pl.ds(c * _L, _L)] for c in range(_D // _L)]
            rv = rows_v.at[buf]

            lo_half = iota < 8
            idx_e = (iota * 2) & 15
            idx_o = idx_e | 1
            ctr_e = [jnp.where(lo_half, vperm(ctr[2 * q], idx_e),
                               vperm(ctr[2 * q + 1], idx_e))
                     for q in range(_D // 32)]
            ctr_o = [jnp.where(lo_half, vperm(ctr[2 * q], idx_o),
                               vperm(ctr[2 * q + 1], idx_o))
                     for q in range(_D // 32)]
            himask = jnp.full((_L,), -65536, jnp.int32)

            def group(g, carry2):
                svec = None
                for r in range(_L):
                    row = g * _L + r
                    prods = []
                    for q in range(_D // 32):
                        w = rv[row, pl.ds(q * _L, _L)]
                        lo = plsc.bitcast(lax.shift_left(w, 16), jnp.float32)
                        hi = plsc.bitcast(w & himask, jnp.float32)
                        prods.append(lo * ctr_e[q])
                        prods.append(hi * ctr_o[q])
                    while len(prods) > 1:
                        prods = [prods[i] + prods[i + 1]
                                 for i in range(0, len(prods), 2)]
                    s = prods[0]
                    for p in perms:
                        s = s + vperm(s, p)
                    svec = s if svec is None else jnp.where(iota == r, s, svec)
                sc_v[buf, pl.ds(pl.multiple_of(g * _L, 8), _L)] = svec
                return carry2

            lax.fori_loop(0, _G, group, 0)

            pltpu.async_copy(
                sc_v.at[buf],
                out_hbm.at[pl.ds(pl.multiple_of((base + b) * _R, 8), _R)],
                sc_sems[buf])

            @pl.when(b + 2 < _BW)
            def _refill():
                issue_gather(b + 2, buf)
        return carry

    lax.fori_loop(0, _BW // 2, step, 0)

    for buf in range(2):
        pltpu.make_async_copy(sc_v.at[buf], out_hbm.at[pl.ds(0, _R)],
                              sc_sems[buf]).wait()


@jax.jit
def _sc_scores(cidx, ctx_idx, tgt, ctx):
    mesh = plsc.VectorSubcoreMesh(core_axis_name="c", subcore_axis_name="s",
                                  num_cores=_NC, num_subcores=_NS)
    f = pl.kernel(
        _sc_body,
        out_type=jax.ShapeDtypeStruct((_B * _R,), jnp.float32),
        mesh=mesh,
        compiler_params=pltpu.CompilerParams(needs_layout_passes=False,
                                             use_tc_tiling_on_sc=False),
        scratch_types=[
            pltpu.VMEM((_BW,), jnp.int32),
            pltpu.VMEM((_BW, _D), jnp.float32),
            pltpu.VMEM((_BW * 2, 128), jnp.int32),
            pltpu.VMEM((2, _R, _D // 2), jnp.int32),
            pltpu.VMEM((2, _R), jnp.float32),
            pltpu.SemaphoreType.DMA,
            pltpu.SemaphoreType.DMA,
            pltpu.SemaphoreType.DMA,
            pltpu.SemaphoreType.DMA,
            pltpu.SemaphoreType.DMA,
        ],
    )
    return f(cidx, ctx_idx, tgt, ctx)


def kernel(center, positive, negative, target_embeddings, context_embeddings):
    center = center.astype(jnp.int32)
    pad = jnp.zeros((_B, _R - _P - _N), jnp.int32)
    ctx_idx = jnp.concatenate(
        [positive.astype(jnp.int32), negative.astype(jnp.int32), pad], axis=1)
    ctx_idx = ctx_idx.reshape(_B * 2, 128)
    ctx_packed = lax.bitcast_convert_type(
        context_embeddings.astype(jnp.bfloat16).reshape(-1, _D // 2, 2),
        jnp.int32)
    scores = _sc_scores(center, ctx_idx, target_embeddings, ctx_packed)
    scores = scores.reshape(_B, _R)
    return scores[:, :_P], scores[:, _P:_P + _N]
